# baseline (speedup 1.0000x reference)
import functools

import jax
import jax.numpy as jnp
from jax import lax
from jax.experimental import pallas as pl
from jax.experimental.pallas import tpu as pltpu
from jax.experimental.pallas import tpu_sc as plsc

NUM_FACTORS = 64
NUM_ROWS = 1000000
BATCH = 16384
NC = 2
NS = 16
L = 16
NW = NC * NS
B_PER_W = BATCH // NW
HALF = B_PER_W // 2
IDX_CHUNK = 128
N_CHUNKS = B_PER_W // IDX_CHUNK

RR = 8192
G = -(-NUM_ROWS // RR)


def _depad_body(x_ref, o_ref):
    xt = x_ref[...].astype(jnp.bfloat16).T
    o_ref[...] = jnp.concatenate([xt[0:RR // 2], xt[RR // 2:RR]], axis=1)


def _to_row_major(table_t):
    return pl.pallas_call(
        _depad_body,
        grid=(G,),
        in_specs=[pl.BlockSpec((NUM_FACTORS, RR), lambda g: (0, g))],
        out_specs=pl.BlockSpec((RR // 2, 128), lambda g: (g, 0)),
        out_shape=jax.ShapeDtypeStruct((G * RR // 2, 128), jnp.bfloat16),
    )(table_t)


def _body(uid_hbm, iid_hbm, ue_hbm, ie_hbm, ub_hbm, ib_hbm, out_hbm,
          idx_u, idx_i, idx2_u, idx2_i, u_rows, i_rows, ub_v, ib_v, out_v, sem):
    wid = lax.axis_index("s") * NC + lax.axis_index("c")
    base = wid * B_PER_W

    for j in range(N_CHUNKS):
        pltpu.sync_copy(uid_hbm.at[pl.ds(base + j * IDX_CHUNK, IDX_CHUNK)],
                        idx_u.at[j])
        pltpu.sync_copy(iid_hbm.at[pl.ds(base + j * IDX_CHUNK, IDX_CHUNK)],
                        idx_i.at[j])

    for j in range(N_CHUNKS):
        for k in range(IDX_CHUNK // L):
            sl = pl.ds(k * L, L)
            tu = idx_u[j, sl]
            idx2_u[j, sl] = (((tu >> 13) << 13) + ((tu & 4095) << 1)
                             + ((tu >> 12) & 1))
            ti = idx_i[j, sl]
            idx2_i[j, sl] = (((ti >> 13) << 13) + ((ti & 4095) << 1)
                             + ((ti >> 12) & 1))

    bias_copies = []
    for j in range(N_CHUNKS):
        sl = pl.ds(j * IDX_CHUNK, IDX_CHUNK)
        bias_copies.append(pltpu.async_copy(ub_hbm.at[idx_u.at[j]],
                                            ub_v.at[sl], sem))
        bias_copies.append(pltpu.async_copy(ib_hbm.at[idx_i.at[j]],
                                            ib_v.at[sl], sem))

    lane = lax.iota(jnp.int32, 16)

    for h in range(2):
        copies = []
        for j in range(2):
            sl = pl.ds(j * IDX_CHUNK, IDX_CHUNK)
            copies.append(pltpu.async_copy(
                ue_hbm.at[idx2_u.at[2 * h + j]], u_rows.at[sl], sem))
            copies.append(pltpu.async_copy(
                ie_hbm.at[idx2_i.at[2 * h + j]], i_rows.at[sl], sem))
        for cpy in copies:
            cpy.wait()

        def chunk_body(c, carry):
            acc = lane * jnp.float32(0)
            for l in range(L):
                r = c * L + l
                ua0, ua1 = plsc.unpack(u_rows[r, pl.ds(0, 32)], format=plsc.PackFormat.INTERLEAVED)
                ub0, ub1 = plsc.unpack(u_rows[r, pl.ds(32, 32)], format=plsc.PackFormat.INTERLEAVED)
                ia0, ia1 = plsc.unpack(i_rows[r, pl.ds(0, 32)], format=plsc.PackFormat.INTERLEAVED)
                ib0, ib1 = plsc.unpack(i_rows[r, pl.ds(32, 32)], format=plsc.PackFormat.INTERLEAVED)
                tot = jnp.sum((ua0 * ia0 + ua1 * ia1) + (ub0 * ib0 + ub1 * ib1))
                acc = jnp.where(lane == l, tot, acc)
            out_v[pl.ds(h * HALF + c * L, L)] = acc
            return carry

        lax.fori_loop(0, HALF // L, chunk_body, 0)

    for cpy in bias_copies:
        cpy.wait()

    def bias_body(c, carry):
        sl = pl.ds(c * L, L)
        out_v[sl] = out_v[sl] + ub_v[sl] + ib_v[sl]
        return carry

    lax.fori_loop(0, B_PER_W // L, bias_body, 0)

    pltpu.sync_copy(out_v, out_hbm.at[pl.ds(base, B_PER_W)])


@jax.jit
def _mf_sc(user_id, item_id, user_embedding, item_embedding, user_bias, item_bias):
    ue2 = _to_row_major(user_embedding.T).reshape(G * RR, NUM_FACTORS)
    ie2 = _to_row_major(item_embedding.T).reshape(G * RR, NUM_FACTORS)
    mesh = plsc.VectorSubcoreMesh(core_axis_name="c", subcore_axis_name="s",
                                  num_cores=NC, num_subcores=NS)
    run = pl.kernel(
        _body,
        out_type=jax.ShapeDtypeStruct((BATCH,), jnp.float32),
        mesh=mesh,
        scratch_types=[
            pltpu.VMEM((N_CHUNKS, IDX_CHUNK), jnp.int32),
            pltpu.VMEM((N_CHUNKS, IDX_CHUNK), jnp.int32),
            pltpu.VMEM((N_CHUNKS, IDX_CHUNK), jnp.int32),
            pltpu.VMEM((N_CHUNKS, IDX_CHUNK), jnp.int32),
            pltpu.VMEM((HALF, NUM_FACTORS), jnp.bfloat16),
            pltpu.VMEM((HALF, NUM_FACTORS), jnp.bfloat16),
            pltpu.VMEM((B_PER_W,), jnp.float32),
            pltpu.VMEM((B_PER_W,), jnp.float32),
            pltpu.VMEM((B_PER_W,), jnp.float32),
            pltpu.SemaphoreType.DMA,
        ],
        compiler_params=pltpu.CompilerParams(needs_layout_passes=False,
                                             use_tc_tiling_on_sc=False),
    )
    return run(user_id, item_id, ue2, ie2,
               user_bias.reshape(-1), item_bias.reshape(-1))


def kernel(user_id, item_id, user_embedding, item_embedding, user_bias, item_bias):
    return _mf_sc(user_id, item_id, user_embedding, item_embedding,
                  user_bias, item_bias)


# speedup vs baseline: 1.0006x; 1.0006x over previous
import functools

import jax
import jax.numpy as jnp
from jax import lax
from jax.experimental import pallas as pl
from jax.experimental.pallas import tpu as pltpu
from jax.experimental.pallas import tpu_sc as plsc

NUM_FACTORS = 64
NUM_ROWS = 1000000
BATCH = 16384
NC = 2
NS = 16
L = 16
NW = NC * NS
B_PER_W = BATCH // NW
HALF = B_PER_W // 2
IDX_CHUNK = 128
N_CHUNKS = B_PER_W // IDX_CHUNK

RR = 8192
G = -(-NUM_ROWS // RR)


def _depad_body(x_ref, o_ref):
    xt = x_ref[...].T.astype(jnp.bfloat16)
    o_ref[...] = jnp.concatenate([xt[0:RR // 2], xt[RR // 2:RR]], axis=1)


def _to_row_major(table_t):
    return pl.pallas_call(
        _depad_body,
        grid=(G,),
        in_specs=[pl.BlockSpec((NUM_FACTORS, RR), lambda g: (0, g))],
        out_specs=pl.BlockSpec((RR // 2, 128), lambda g: (g, 0)),
        out_shape=jax.ShapeDtypeStruct((G * RR // 2, 128), jnp.bfloat16),
    )(table_t)


def _body(uid_hbm, iid_hbm, ue_hbm, ie_hbm, ub_hbm, ib_hbm, out_hbm,
          idx_u, idx_i, idx2_u, idx2_i, u_rows, i_rows, ub_v, ib_v, out_v, sem):
    wid = lax.axis_index("s") * NC + lax.axis_index("c")
    base = wid * B_PER_W

    for j in range(N_CHUNKS):
        pltpu.sync_copy(uid_hbm.at[pl.ds(base + j * IDX_CHUNK, IDX_CHUNK)],
                        idx_u.at[j])
        pltpu.sync_copy(iid_hbm.at[pl.ds(base + j * IDX_CHUNK, IDX_CHUNK)],
                        idx_i.at[j])

    for j in range(N_CHUNKS):
        for k in range(IDX_CHUNK // L):
            sl = pl.ds(k * L, L)
            tu = idx_u[j, sl]
            idx2_u[j, sl] = (((tu >> 13) << 13) + ((tu & 4095) << 1)
                             + ((tu >> 12) & 1))
            ti = idx_i[j, sl]
            idx2_i[j, sl] = (((ti >> 13) << 13) + ((ti & 4095) << 1)
                             + ((ti >> 12) & 1))

    bias_copies = []
    for j in range(N_CHUNKS):
        sl = pl.ds(j * IDX_CHUNK, IDX_CHUNK)
        bias_copies.append(pltpu.async_copy(ub_hbm.at[idx_u.at[j]],
                                            ub_v.at[sl], sem))
        bias_copies.append(pltpu.async_copy(ib_hbm.at[idx_i.at[j]],
                                            ib_v.at[sl], sem))

    lane = lax.iota(jnp.int32, 16)

    for h in range(2):
        copies = []
        for j in range(2):
            sl = pl.ds(j * IDX_CHUNK, IDX_CHUNK)
            copies.append(pltpu.async_copy(
                ue_hbm.at[idx2_u.at[2 * h + j]], u_rows.at[sl], sem))
            copies.append(pltpu.async_copy(
                ie_hbm.at[idx2_i.at[2 * h + j]], i_rows.at[sl], sem))
        for cpy in copies:
            cpy.wait()

        def chunk_body(c, carry):
            acc = lane * jnp.float32(0)
            for l in range(L):
                r = c * L + l
                ua0, ua1 = plsc.unpack(u_rows[r, pl.ds(0, 32)], format=plsc.PackFormat.INTERLEAVED)
                ub0, ub1 = plsc.unpack(u_rows[r, pl.ds(32, 32)], format=plsc.PackFormat.INTERLEAVED)
                ia0, ia1 = plsc.unpack(i_rows[r, pl.ds(0, 32)], format=plsc.PackFormat.INTERLEAVED)
                ib0, ib1 = plsc.unpack(i_rows[r, pl.ds(32, 32)], format=plsc.PackFormat.INTERLEAVED)
                tot = jnp.sum((ua0 * ia0 + ua1 * ia1) + (ub0 * ib0 + ub1 * ib1))
                acc = jnp.where(lane == l, tot, acc)
            out_v[pl.ds(h * HALF + c * L, L)] = acc
            return carry

        lax.fori_loop(0, HALF // L, chunk_body, 0)

    for cpy in bias_copies:
        cpy.wait()

    def bias_body(c, carry):
        sl = pl.ds(c * L, L)
        out_v[sl] = out_v[sl] + ub_v[sl] + ib_v[sl]
        return carry

    lax.fori_loop(0, B_PER_W // L, bias_body, 0)

    pltpu.sync_copy(out_v, out_hbm.at[pl.ds(base, B_PER_W)])


@jax.jit
def _mf_sc(user_id, item_id, user_embedding, item_embedding, user_bias, item_bias):
    ue2 = _to_row_major(user_embedding.T).reshape(G * RR, NUM_FACTORS)
    ie2 = _to_row_major(item_embedding.T).reshape(G * RR, NUM_FACTORS)
    mesh = plsc.VectorSubcoreMesh(core_axis_name="c", subcore_axis_name="s",
                                  num_cores=NC, num_subcores=NS)
    run = pl.kernel(
        _body,
        out_type=jax.ShapeDtypeStruct((BATCH,), jnp.float32),
        mesh=mesh,
        scratch_types=[
            pltpu.VMEM((N_CHUNKS, IDX_CHUNK), jnp.int32),
            pltpu.VMEM((N_CHUNKS, IDX_CHUNK), jnp.int32),
            pltpu.VMEM((N_CHUNKS, IDX_CHUNK), jnp.int32),
            pltpu.VMEM((N_CHUNKS, IDX_CHUNK), jnp.int32),
            pltpu.VMEM((HALF, NUM_FACTORS), jnp.bfloat16),
            pltpu.VMEM((HALF, NUM_FACTORS), jnp.bfloat16),
            pltpu.VMEM((B_PER_W,), jnp.float32),
            pltpu.VMEM((B_PER_W,), jnp.float32),
            pltpu.VMEM((B_PER_W,), jnp.float32),
            pltpu.SemaphoreType.DMA,
        ],
        compiler_params=pltpu.CompilerParams(needs_layout_passes=False,
                                             use_tc_tiling_on_sc=False),
    )
    return run(user_id, item_id, ue2, ie2,
               user_bias.reshape(-1), item_bias.reshape(-1))


def kernel(user_id, item_id, user_embedding, item_embedding, user_bias, item_bias):
    return _mf_sc(user_id, item_id, user_embedding, item_embedding,
                  user_bias, item_bias)


# speedup vs baseline: 2.1317x; 2.1304x over previous
import functools

import jax
import jax.numpy as jnp
from jax import lax
from jax.experimental import pallas as pl
from jax.experimental.pallas import tpu as pltpu
from jax.experimental.pallas import tpu_sc as plsc

NUM_FACTORS = 64
NUM_ROWS = 1000000
BATCH = 16384
NC = 2
NS = 16
L = 16
NW = NC * NS
B_PER_W = BATCH // NW
HALF = B_PER_W // 2
IDX_CHUNK = 128
N_CHUNKS = B_PER_W // IDX_CHUNK

RR = 16384
G = -(-NUM_ROWS // RR)


def _depad_body(x_ref, o_ref):
    xt = x_ref[...].T
    o_ref[...] = jnp.concatenate([xt[0:RR // 2], xt[RR // 2:RR]], axis=1)


def _to_row_major(table_t):
    return pl.pallas_call(
        _depad_body,
        grid=(G,),
        in_specs=[pl.BlockSpec((NUM_FACTORS, RR), lambda g: (0, g))],
        out_specs=pl.BlockSpec((RR // 2, 128), lambda g: (g, 0)),
        out_shape=jax.ShapeDtypeStruct((G * RR // 2, 128), jnp.float32),
    )(table_t)


def _body(uid_hbm, iid_hbm, ue_hbm, ie_hbm, ub_hbm, ib_hbm, out_hbm,
          idx_u, idx_i, idx2_u, idx2_i, u_rows, i_rows, ub_v, ib_v, out_v, sem):
    wid = lax.axis_index("s") * NC + lax.axis_index("c")
    base = wid * B_PER_W

    for j in range(N_CHUNKS):
        pltpu.sync_copy(uid_hbm.at[pl.ds(base + j * IDX_CHUNK, IDX_CHUNK)],
                        idx_u.at[j])
        pltpu.sync_copy(iid_hbm.at[pl.ds(base + j * IDX_CHUNK, IDX_CHUNK)],
                        idx_i.at[j])

    for j in range(N_CHUNKS):
        for k in range(IDX_CHUNK // L):
            sl = pl.ds(k * L, L)
            tu = idx_u[j, sl]
            idx2_u[j, sl] = (((tu >> 14) << 14) + ((tu & 8191) << 1)
                             + ((tu >> 13) & 1))
            ti = idx_i[j, sl]
            idx2_i[j, sl] = (((ti >> 14) << 14) + ((ti & 8191) << 1)
                             + ((ti >> 13) & 1))

    bias_copies = []
    for j in range(N_CHUNKS):
        sl = pl.ds(j * IDX_CHUNK, IDX_CHUNK)
        bias_copies.append(pltpu.async_copy(ub_hbm.at[idx_u.at[j]],
                                            ub_v.at[sl], sem))
        bias_copies.append(pltpu.async_copy(ib_hbm.at[idx_i.at[j]],
                                            ib_v.at[sl], sem))

    lane = lax.iota(jnp.int32, 16)

    for h in range(2):
        copies = []
        for j in range(2):
            sl = pl.ds(j * IDX_CHUNK, IDX_CHUNK)
            copies.append(pltpu.async_copy(
                ue_hbm.at[idx2_u.at[2 * h + j]], u_rows.at[sl], sem))
            copies.append(pltpu.async_copy(
                ie_hbm.at[idx2_i.at[2 * h + j]], i_rows.at[sl], sem))
        for cpy in copies:
            cpy.wait()

        def chunk_body(c, carry):
            acc = lane * jnp.float32(0)
            for l in range(L):
                r = c * L + l
                q0 = u_rows[r, pl.ds(0, 16)] * i_rows[r, pl.ds(0, 16)]
                q1 = u_rows[r, pl.ds(16, 16)] * i_rows[r, pl.ds(16, 16)]
                q2 = u_rows[r, pl.ds(32, 16)] * i_rows[r, pl.ds(32, 16)]
                q3 = u_rows[r, pl.ds(48, 16)] * i_rows[r, pl.ds(48, 16)]
                tot = jnp.sum((q0 + q1) + (q2 + q3))
                acc = jnp.where(lane == l, tot, acc)
            out_v[pl.ds(h * HALF + c * L, L)] = acc
            return carry

        lax.fori_loop(0, HALF // L, chunk_body, 0)

    for cpy in bias_copies:
        cpy.wait()

    def bias_body(c, carry):
        sl = pl.ds(c * L, L)
        out_v[sl] = out_v[sl] + ub_v[sl] + ib_v[sl]
        return carry

    lax.fori_loop(0, B_PER_W // L, bias_body, 0)

    pltpu.sync_copy(out_v, out_hbm.at[pl.ds(base, B_PER_W)])


@jax.jit
def _mf_sc(user_id, item_id, user_embedding, item_embedding, user_bias, item_bias):
    ue2 = _to_row_major(user_embedding.T).reshape(G * RR, NUM_FACTORS)
    ie2 = _to_row_major(item_embedding.T).reshape(G * RR, NUM_FACTORS)
    mesh = plsc.VectorSubcoreMesh(core_axis_name="c", subcore_axis_name="s",
                                  num_cores=NC, num_subcores=NS)
    run = pl.kernel(
        _body,
        out_type=jax.ShapeDtypeStruct((BATCH,), jnp.float32),
        mesh=mesh,
        scratch_types=[
            pltpu.VMEM((N_CHUNKS, IDX_CHUNK), jnp.int32),
            pltpu.VMEM((N_CHUNKS, IDX_CHUNK), jnp.int32),
            pltpu.VMEM((N_CHUNKS, IDX_CHUNK), jnp.int32),
            pltpu.VMEM((N_CHUNKS, IDX_CHUNK), jnp.int32),
            pltpu.VMEM((HALF, NUM_FACTORS), jnp.float32),
            pltpu.VMEM((HALF, NUM_FACTORS), jnp.float32),
            pltpu.VMEM((B_PER_W,), jnp.float32),
            pltpu.VMEM((B_PER_W,), jnp.float32),
            pltpu.VMEM((B_PER_W,), jnp.float32),
            pltpu.SemaphoreType.DMA,
        ],
        compiler_params=pltpu.CompilerParams(needs_layout_passes=False,
                                             use_tc_tiling_on_sc=False),
    )
    return run(user_id, item_id, ue2, ie2,
               user_bias.reshape(-1), item_bias.reshape(-1))


def kernel(user_id, item_id, user_embedding, item_embedding, user_bias, item_bias):
    return _mf_sc(user_id, item_id, user_embedding, item_embedding,
                  user_bias, item_bias)


# speedup vs baseline: 2.2456x; 1.0534x over previous
import functools

import jax
import jax.numpy as jnp
from jax import lax
from jax.experimental import pallas as pl
from jax.experimental.pallas import tpu as pltpu
from jax.experimental.pallas import tpu_sc as plsc

NUM_FACTORS = 64
NUM_ROWS = 1000000
BATCH = 16384
NC = 2
NS = 16
L = 16
NW = NC * NS
B_PER_W = BATCH // NW
HALF = B_PER_W // 2
IDX_CHUNK = 128
N_CHUNKS = B_PER_W // IDX_CHUNK

RR = 32768
G = -(-NUM_ROWS // RR)


def _depad_body(x_ref, o_ref):
    xt = x_ref[...].T
    o_ref[...] = jnp.concatenate([xt[0:RR // 2], xt[RR // 2:RR]], axis=1)


def _to_row_major(table_t):
    return pl.pallas_call(
        _depad_body,
        grid=(G,),
        in_specs=[pl.BlockSpec((NUM_FACTORS, RR), lambda g: (0, g))],
        out_specs=pl.BlockSpec((RR // 2, 128), lambda g: (g, 0)),
        out_shape=jax.ShapeDtypeStruct((G * RR // 2, 128), jnp.float32),
    )(table_t)


def _body(uid_hbm, iid_hbm, ue_hbm, ie_hbm, ub_hbm, ib_hbm, out_hbm,
          idx_u, idx_i, idx2_u, idx2_i, u_rows, i_rows, ub_v, ib_v, out_v, sem):
    wid = lax.axis_index("s") * NC + lax.axis_index("c")
    base = wid * B_PER_W

    for j in range(N_CHUNKS):
        pltpu.sync_copy(uid_hbm.at[pl.ds(base + j * IDX_CHUNK, IDX_CHUNK)],
                        idx_u.at[j])
        pltpu.sync_copy(iid_hbm.at[pl.ds(base + j * IDX_CHUNK, IDX_CHUNK)],
                        idx_i.at[j])

    for j in range(N_CHUNKS):
        for k in range(IDX_CHUNK // L):
            sl = pl.ds(k * L, L)
            tu = idx_u[j, sl]
            idx2_u[j, sl] = (((tu >> 15) << 15) + ((tu & 16383) << 1)
                             + ((tu >> 14) & 1))
            ti = idx_i[j, sl]
            idx2_i[j, sl] = (((ti >> 15) << 15) + ((ti & 16383) << 1)
                             + ((ti >> 14) & 1))

    bias_copies = []
    for j in range(N_CHUNKS):
        sl = pl.ds(j * IDX_CHUNK, IDX_CHUNK)
        bias_copies.append(pltpu.async_copy(ub_hbm.at[idx_u.at[j]],
                                            ub_v.at[sl], sem))
        bias_copies.append(pltpu.async_copy(ib_hbm.at[idx_i.at[j]],
                                            ib_v.at[sl], sem))

    lane = lax.iota(jnp.int32, 16)

    for h in range(2):
        copies = []
        for j in range(2):
            sl = pl.ds(j * IDX_CHUNK, IDX_CHUNK)
            copies.append(pltpu.async_copy(
                ue_hbm.at[idx2_u.at[2 * h + j]], u_rows.at[sl], sem))
            copies.append(pltpu.async_copy(
                ie_hbm.at[idx2_i.at[2 * h + j]], i_rows.at[sl], sem))
        for cpy in copies:
            cpy.wait()

        def chunk_body(c, carry):
            acc = lane * jnp.float32(0)
            for l in range(L):
                r = c * L + l
                q0 = u_rows[r, pl.ds(0, 16)] * i_rows[r, pl.ds(0, 16)]
                q1 = u_rows[r, pl.ds(16, 16)] * i_rows[r, pl.ds(16, 16)]
                q2 = u_rows[r, pl.ds(32, 16)] * i_rows[r, pl.ds(32, 16)]
                q3 = u_rows[r, pl.ds(48, 16)] * i_rows[r, pl.ds(48, 16)]
                tot = jnp.sum((q0 + q1) + (q2 + q3))
                acc = jnp.where(lane == l, tot, acc)
            out_v[pl.ds(h * HALF + c * L, L)] = acc
            return carry

        lax.fori_loop(0, HALF // L, chunk_body, 0)

    for cpy in bias_copies:
        cpy.wait()

    def bias_body(c, carry):
        sl = pl.ds(c * L, L)
        out_v[sl] = out_v[sl] + ub_v[sl] + ib_v[sl]
        return carry

    lax.fori_loop(0, B_PER_W // L, bias_body, 0)

    pltpu.sync_copy(out_v, out_hbm.at[pl.ds(base, B_PER_W)])


@jax.jit
def _mf_sc(user_id, item_id, user_embedding, item_embedding, user_bias, item_bias):
    ue2 = _to_row_major(user_embedding.T).reshape(G * RR, NUM_FACTORS)
    ie2 = _to_row_major(item_embedding.T).reshape(G * RR, NUM_FACTORS)
    mesh = plsc.VectorSubcoreMesh(core_axis_name="c", subcore_axis_name="s",
                                  num_cores=NC, num_subcores=NS)
    run = pl.kernel(
        _body,
        out_type=jax.ShapeDtypeStruct((BATCH,), jnp.float32),
        mesh=mesh,
        scratch_types=[
            pltpu.VMEM((N_CHUNKS, IDX_CHUNK), jnp.int32),
            pltpu.VMEM((N_CHUNKS, IDX_CHUNK), jnp.int32),
            pltpu.VMEM((N_CHUNKS, IDX_CHUNK), jnp.int32),
            pltpu.VMEM((N_CHUNKS, IDX_CHUNK), jnp.int32),
            pltpu.VMEM((HALF, NUM_FACTORS), jnp.float32),
            pltpu.VMEM((HALF, NUM_FACTORS), jnp.float32),
            pltpu.VMEM((B_PER_W,), jnp.float32),
            pltpu.VMEM((B_PER_W,), jnp.float32),
            pltpu.VMEM((B_PER_W,), jnp.float32),
            pltpu.SemaphoreType.DMA,
        ],
        compiler_params=pltpu.CompilerParams(needs_layout_passes=False,
                                             use_tc_tiling_on_sc=False),
    )
    return run(user_id, item_id, ue2, ie2,
               user_bias.reshape(-1), item_bias.reshape(-1))


def kernel(user_id, item_id, user_embedding, item_embedding, user_bias, item_bias):
    return _mf_sc(user_id, item_id, user_embedding, item_embedding,
                  user_bias, item_bias)
